# baseline (speedup 1.0000x reference)
import functools

import jax
import jax.numpy as jnp
from jax import lax
from jax.experimental import pallas as pl
from jax.experimental.pallas import tpu as pltpu
from jax.experimental.pallas import tpu_sc as plsc

_B, _T, _D = 4096, 125, 128
_V = 1000
_NC, _NS = 2, 16
_NW = _NC * _NS
_BPW = _B // _NW
_LANES = 16
_DV = _D // _LANES
_NBUF = 4

_mesh = plsc.VectorSubcoreMesh(
    core_axis_name="c", subcore_axis_name="s", num_cores=_NC, num_subcores=_NS
)


@functools.partial(
    pl.kernel,
    out_type=(
        jax.ShapeDtypeStruct((_T, _B, _D), jnp.float32),
        jax.ShapeDtypeStruct((_B, 1, _D), jnp.float32),
        jax.ShapeDtypeStruct((_B, 1, _D), jnp.float32),
    ),
    mesh=_mesh,
    scratch_types=[
        pltpu.VMEM_SHARED((_V, _D), jnp.float32),
        pltpu.VMEM((_T, _BPW), jnp.int32),
        pltpu.VMEM((_T, _D), jnp.float32),
        pltpu.VMEM((3, _D), jnp.float32),
        [pltpu.VMEM((_BPW, _D), jnp.float32)] * _NBUF,
        pltpu.VMEM((_BPW, 1, _D), jnp.float32),
        [pltpu.SemaphoreType.DMA] * _NBUF,
        [pltpu.SemaphoreType.DMA] * _NBUF,
    ],
)
def _embed_sc(zeo, syn, idxt_hbm, table, te_hbm, pe_hbm,
              out_seq, out_zeo, out_syn,
              table_sh, idx_v, bias_v, te_v, rows, zs_v, gsem, osem):
    sid = lax.axis_index("s")
    wid = sid * _NC + lax.axis_index("c")
    base = wid * _BPW

    @pl.when(sid == 0)
    def _():
        pltpu.sync_copy(table, table_sh)
    plsc.subcore_barrier()

    pltpu.sync_copy(te_hbm, te_v)
    pltpu.sync_copy(pe_hbm, bias_v)
    pltpu.sync_copy(idxt_hbm.at[:, pl.ds(base, _BPW)], idx_v)

    def bias_body(t5, carry):
        for u in range(5):
            t = t5 * 5 + u
            for d in range(_DV):
                sl = pl.ds(d * _LANES, _LANES)
                plsc.addupdate(bias_v.at[t, sl], te_v[2, sl])
        return carry
    lax.fori_loop(0, _T // 5, bias_body, 0)

    for src, dst, row in ((zeo, out_zeo, 0), (syn, out_syn, 1)):
        pltpu.sync_copy(src.at[pl.ds(base, _BPW)], zs_v)
        def zs_body(i, carry, row=row):
            for d in range(_DV):
                sl = pl.ds(d * _LANES, _LANES)
                plsc.addupdate(zs_v.at[i, 0, sl], te_v[row, sl])
            return carry
        lax.fori_loop(0, _BPW, zs_body, 0)
        pltpu.sync_copy(zs_v, dst.at[pl.ds(base, _BPW)])

    plsc.subcore_barrier()

    def g_copy(t, j):
        return pltpu.make_async_copy(
            table_sh.at[idx_v.at[t]], rows[j], gsem[j])

    def o_copy(t, j):
        return pltpu.make_async_copy(
            rows[j], out_seq.at[t, pl.ds(base, _BPW)], osem[j])

    def add_bias(t, j):
        bias_regs = [bias_v[t, pl.ds(d * _LANES, _LANES)] for d in range(_DV)]
        def add_body(r4, carry):
            for rr in range(4):
                r = r4 * 4 + rr
                for d in range(_DV):
                    sl = pl.ds(d * _LANES, _LANES)
                    plsc.addupdate(rows[j].at[r, sl], bias_regs[d])
            return carry
        lax.fori_loop(0, _BPW // 4, add_body, 0)

    g_copy(0, 0).start()
    g_copy(1, 1).start()
    for t in (0, 1):
        g_copy(t, t).wait()
        add_bias(t, t)
        o_copy(t, t).start()
        g_copy(t + 2, t + 2).start()

    def main_body(t4, carry):
        for j in range(_NBUF):
            t = 2 + t4 * _NBUF + j
            buf = (2 + j) % _NBUF
            nbuf = j % _NBUF
            g_copy(t, buf).wait()
            add_bias(t, buf)
            o_copy(t, buf).start()
            o_copy(t - 2, nbuf).wait()
            g_copy(t + 2, nbuf).start()
        return carry
    lax.fori_loop(0, (_T - 5) // _NBUF, main_body, 0)

    o_copy(120, 0).wait()
    g_copy(124, 0).start()
    for t in (122, 123, 124):
        j = t % _NBUF
        g_copy(t, j).wait()
        add_bias(t, j)
        o_copy(t, j).start()
    for t in range(_T - _NBUF, _T):
        o_copy(t, t % _NBUF).wait()


def kernel(zeo, syn, smis_seq, char_embed, type_embed, pe):
    idx_t = smis_seq.astype(jnp.int32).T
    pe2d = pe.reshape(_T, _D)
    out_t, out_zeo, out_syn = _embed_sc(
        zeo, syn, idx_t, char_embed, type_embed, pe2d)
    return out_t.transpose(1, 0, 2), out_zeo, out_syn
